# baseline (speedup 1.0000x reference)
import functools

import jax
import jax.numpy as jnp
from jax import lax
from jax.experimental import pallas as pl
from jax.experimental.pallas import tpu as pltpu
from jax.experimental.pallas import tpu_sc as plsc

N = 4096
E = 16384
IN_F = 256
OUT_F = 128

def _k_support(p_ref, x_ref, w_ref, sm_ref, prod_ref):
    p2 = 2.0 * jax.nn.sigmoid(p_ref[...])
    support = jnp.dot(x_ref[...], w_ref[...], preferred_element_type=jnp.float32)
    e = support * p2
    sm = jnp.exp(e - jnp.max(e))
    sm_ref[...] = sm
    prod_ref[...] = support * sm


def _support_sm_prod(p, x, w):
    return pl.pallas_call(
        _k_support,
        out_shape=(
            jax.ShapeDtypeStruct((N, OUT_F), jnp.float32),
            jax.ShapeDtypeStruct((N, OUT_F), jnp.float32),
        ),
    )(p, x, w)


_RB2 = 512

def _k_agg(adj_ref, sm_ref, agg_ref):
    agg_ref[...] = jnp.dot(adj_ref[...], sm_ref[...],
                           preferred_element_type=jnp.float32)


def _agg(adj, sm):
    grid = (N // _RB2,)
    return pl.pallas_call(
        _k_agg,
        grid=grid,
        in_specs=[
            pl.BlockSpec((_RB2, N), lambda i: (i, 0)),
            pl.BlockSpec((N, OUT_F), lambda i: (0, 0)),
        ],
        out_specs=pl.BlockSpec((_RB2, OUT_F), lambda i: (i, 0)),
        out_shape=jax.ShapeDtypeStruct((N, OUT_F), jnp.float32),
        compiler_params=pltpu.CompilerParams(
            dimension_semantics=("arbitrary",)),
    )(adj, sm)


_NC = 2
_NS = 16
_NW = _NC * _NS
_EPW = E // _NW
_CHUNK = 128
_NCHUNK = _EPW // _CHUNK


def _gather_one(table, idx):
    mesh = plsc.VectorSubcoreMesh(core_axis_name="c", subcore_axis_name="s")

    @functools.partial(
        pl.kernel,
        mesh=mesh,
        out_type=jax.ShapeDtypeStruct((E, OUT_F), jnp.float32),
        scratch_types=[
            pltpu.VMEM((_CHUNK,), jnp.int32),
            pltpu.VMEM((_CHUNK,), jnp.int32),
            pltpu.VMEM((_CHUNK, OUT_F), jnp.float32),
            pltpu.VMEM((_CHUNK, OUT_F), jnp.float32),
            pltpu.SemaphoreType.DMA,
            pltpu.SemaphoreType.DMA,
        ],
    )
    def k(tab_hbm, idx_hbm, out_hbm, idx_a, idx_b, rows_a, rows_b, sem_a, sem_b):
        wid = lax.axis_index("s") * _NC + lax.axis_index("c")
        base = wid * _EPW
        pltpu.sync_copy(idx_hbm.at[pl.ds(base, _CHUNK)], idx_a)
        cp = pltpu.async_copy(tab_hbm.at[idx_a], rows_a, sem_a)
        for c in range(_NCHUNK):
            idx_n = idx_b if c % 2 == 0 else idx_a
            rows_n = rows_b if c % 2 == 0 else rows_a
            sem_n = sem_b if c % 2 == 0 else sem_a
            rows_c = rows_a if c % 2 == 0 else rows_b
            if c + 1 < _NCHUNK:
                off_n = base + (c + 1) * _CHUNK
                pltpu.sync_copy(idx_hbm.at[pl.ds(off_n, _CHUNK)], idx_n)
                cp_n = pltpu.async_copy(tab_hbm.at[idx_n], rows_n, sem_n)
            cp.wait()
            pltpu.sync_copy(rows_c, out_hbm.at[pl.ds(base + c * _CHUNK, _CHUNK)])
            if c + 1 < _NCHUNK:
                cp = cp_n

    return k(table, idx)


_KB = 1024

def _k_out(t_ref, gp_ref, ga_ref, b_ref, out_ref):
    j = pl.program_id(0)
    msg = gp_ref[...] / (ga_ref[...] + 1e-6)
    part = jnp.dot(t_ref[...], msg, preferred_element_type=jnp.float32)

    @pl.when(j == 0)
    def _():
        out_ref[...] = part + b_ref[...]

    @pl.when(j > 0)
    def _():
        out_ref[...] = out_ref[...] + part


def _final(T, gp, ga, bias):
    grid = (E // _KB,)
    return pl.pallas_call(
        _k_out,
        grid=grid,
        in_specs=[
            pl.BlockSpec((N, _KB), lambda j: (0, j)),
            pl.BlockSpec((_KB, OUT_F), lambda j: (j, 0)),
            pl.BlockSpec((_KB, OUT_F), lambda j: (j, 0)),
            pl.BlockSpec((1, OUT_F), lambda j: (0, 0)),
        ],
        out_specs=pl.BlockSpec((N, OUT_F), lambda j: (0, 0)),
        out_shape=jax.ShapeDtypeStruct((N, OUT_F), jnp.float32),
        compiler_params=pltpu.CompilerParams(
            dimension_semantics=("arbitrary",)),
    )(T, gp, ga, bias)


def kernel(input, T, adj, edge, p, weight, bias):
    p11 = p.reshape(1, 1)
    sm, prod = _support_sm_prod(p11, input, weight)
    gp = _gather_one(prod, edge[1])
    agg = _agg(adj, sm)
    ga = _gather_one(agg, edge[0])
    return _final(T, gp, ga, bias.reshape(1, OUT_F))


# speedup vs baseline: 1.0634x; 1.0634x over previous
import functools

import jax
import jax.numpy as jnp
from jax import lax
from jax.experimental import pallas as pl
from jax.experimental.pallas import tpu as pltpu
from jax.experimental.pallas import tpu_sc as plsc

N = 4096
E = 16384
IN_F = 256
OUT_F = 128

_RB2 = 512

def _k_sup_agg(p_ref, x_ref, w_ref, adj_ref, prod_ref, agg_ref, sm_s):
    @pl.when(pl.program_id(0) == 0)
    def _():
        p2 = 2.0 * jax.nn.sigmoid(p_ref[...])
        support = jnp.dot(x_ref[...], w_ref[...],
                          preferred_element_type=jnp.float32)
        e = support * p2
        sm = jnp.exp(e - jnp.max(e))
        sm_s[...] = sm
        prod_ref[...] = support * sm

    agg_ref[...] = jnp.dot(adj_ref[...], sm_s[...],
                           preferred_element_type=jnp.float32)


def _sup_agg(p, x, w, adj):
    grid = (N // _RB2,)
    return pl.pallas_call(
        _k_sup_agg,
        grid=grid,
        in_specs=[
            pl.BlockSpec((1, 1), lambda i: (0, 0)),
            pl.BlockSpec((N, IN_F), lambda i: (0, 0)),
            pl.BlockSpec((IN_F, OUT_F), lambda i: (0, 0)),
            pl.BlockSpec((_RB2, N), lambda i: (i, 0)),
        ],
        out_specs=(
            pl.BlockSpec((N, OUT_F), lambda i: (0, 0)),
            pl.BlockSpec((_RB2, OUT_F), lambda i: (i, 0)),
        ),
        out_shape=(
            jax.ShapeDtypeStruct((N, OUT_F), jnp.float32),
            jax.ShapeDtypeStruct((N, OUT_F), jnp.float32),
        ),
        scratch_shapes=[pltpu.VMEM((N, OUT_F), jnp.float32)],
        compiler_params=pltpu.CompilerParams(
            dimension_semantics=("arbitrary",)),
    )(p, x, w, adj)


_NC = 2
_NS = 16
_NW = _NC * _NS
_EPW = E // _NW
_CHUNK = 128
_NCHUNK = _EPW // _CHUNK


def _gather_two(prod, agg, e1, e0):
    mesh = plsc.VectorSubcoreMesh(core_axis_name="c", subcore_axis_name="s")

    @functools.partial(
        pl.kernel,
        mesh=mesh,
        out_type=(
            jax.ShapeDtypeStruct((E, OUT_F), jnp.float32),
            jax.ShapeDtypeStruct((E, OUT_F), jnp.float32),
        ),
        scratch_types=[
            pltpu.VMEM((2, _CHUNK), jnp.int32),
            pltpu.VMEM((2, _CHUNK), jnp.int32),
            pltpu.VMEM((2, _CHUNK, OUT_F), jnp.float32),
            pltpu.VMEM((2, _CHUNK, OUT_F), jnp.float32),
            pltpu.SemaphoreType.DMA,
            pltpu.SemaphoreType.DMA,
            pltpu.SemaphoreType.DMA,
            pltpu.SemaphoreType.DMA,
        ],
    )
    def k(prod_hbm, agg_hbm, e1_hbm, e0_hbm, gp_hbm, ga_hbm,
          idx1_v, idx0_v, r1_v, r0_v, s1a, s1b, s0a, s0b):
        wid = lax.axis_index("s") * _NC + lax.axis_index("c")
        base = wid * _EPW
        sems1 = (s1a, s1b)
        sems0 = (s0a, s0b)

        def start(c, slot):
            off = base + c * _CHUNK
            pltpu.sync_copy(e1_hbm.at[pl.ds(off, _CHUNK)], idx1_v.at[slot])
            pltpu.sync_copy(e0_hbm.at[pl.ds(off, _CHUNK)], idx0_v.at[slot])
            cp1 = pltpu.async_copy(prod_hbm.at[idx1_v.at[slot]], r1_v.at[slot],
                                   sems1[slot])
            cp0 = pltpu.async_copy(agg_hbm.at[idx0_v.at[slot]], r0_v.at[slot],
                                   sems0[slot])
            return cp1, cp0

        cur = start(0, 0)
        for c in range(_NCHUNK):
            slot = c % 2
            nxt = start(c + 1, 1 - slot) if c + 1 < _NCHUNK else None
            cur[0].wait()
            cur[1].wait()
            off = base + c * _CHUNK
            pltpu.sync_copy(r1_v.at[slot], gp_hbm.at[pl.ds(off, _CHUNK)])
            pltpu.sync_copy(r0_v.at[slot], ga_hbm.at[pl.ds(off, _CHUNK)])
            cur = nxt

    return k(prod, agg, e1, e0)


_KB = 1024

def _k_out(t_ref, gp_ref, ga_ref, b_ref, out_ref):
    j = pl.program_id(0)
    msg = gp_ref[...] / (ga_ref[...] + 1e-6)
    part = jnp.dot(t_ref[...], msg, preferred_element_type=jnp.float32)

    @pl.when(j == 0)
    def _():
        out_ref[...] = part + b_ref[...]

    @pl.when(j > 0)
    def _():
        out_ref[...] = out_ref[...] + part


def _final(T, gp, ga, bias):
    grid = (E // _KB,)
    return pl.pallas_call(
        _k_out,
        grid=grid,
        in_specs=[
            pl.BlockSpec((N, _KB), lambda j: (0, j)),
            pl.BlockSpec((_KB, OUT_F), lambda j: (j, 0)),
            pl.BlockSpec((_KB, OUT_F), lambda j: (j, 0)),
            pl.BlockSpec((1, OUT_F), lambda j: (0, 0)),
        ],
        out_specs=pl.BlockSpec((N, OUT_F), lambda j: (0, 0)),
        out_shape=jax.ShapeDtypeStruct((N, OUT_F), jnp.float32),
        compiler_params=pltpu.CompilerParams(
            dimension_semantics=("arbitrary",)),
    )(T, gp, ga, bias)


def kernel(input, T, adj, edge, p, weight, bias):
    p11 = p.reshape(1, 1)
    prod, agg = _sup_agg(p11, input, weight, adj)
    gp, ga = _gather_two(prod, agg, edge[1], edge[0])
    return _final(T, gp, ga, bias.reshape(1, OUT_F))
